# baseline (speedup 1.0000x reference)
import functools

import jax
import jax.numpy as jnp
from jax import lax
from jax.experimental import pallas as pl
from jax.experimental.pallas import tpu as pltpu
from jax.experimental.pallas import tpu_sc as plsc

_DIM = 16
_G = 128
_GPC = 8


@functools.lru_cache(maxsize=None)
def _build_gather(b_batch: int, l_seq: int, vocab: int):
    info = plsc.get_sparse_core_info()
    nc, ns = info.num_cores, info.num_subcores
    nw = nc * ns
    total = b_batch * l_seq
    npw = total // nw
    rows = _GPC * _G
    assert npw * nw == total and npw % rows == 0
    assert b_batch % rows == 0 and vocab % 8 == 0
    ng = npw // _G
    rstride = rows + 8

    mesh = plsc.VectorSubcoreMesh(core_axis_name="c", subcore_axis_name="s")

    @functools.partial(
        pl.kernel,
        mesh=mesh,
        compiler_params=pltpu.CompilerParams(
            use_tc_tiling_on_sc=True, needs_layout_passes=False),
        out_type=[
            jax.ShapeDtypeStruct((l_seq * _DIM * b_batch,), jnp.float32),
            jax.ShapeDtypeStruct((l_seq * _DIM * b_batch,), jnp.float32),
        ],
        scratch_types=[
            pltpu.VMEM((ng, _G), jnp.int32),
            pltpu.VMEM((2, _G), jnp.int32),
            pltpu.VMEM((2, _G, 128), jnp.float32),
            pltpu.VMEM((2, _G, 128), jnp.float32),
            pltpu.VMEM((_DIM * rstride,), jnp.float32),
            pltpu.VMEM((_DIM * rstride,), jnp.float32),
            pltpu.SemaphoreType.DMA,
            pltpu.SemaphoreType.DMA,
            pltpu.SemaphoreType.DMA,
        ],
    )
    def gather_kernel(ids_hbm, er_hbm, ei_hbm, out_r, out_i,
                      idx_v, blk_idx, blk_r, blk_i, pr_v, pi_v,
                      sem_r, sem_i, sem_o):
        wid = lax.axis_index("s") * nc + lax.axis_index("c")
        pltpu.sync_copy(ids_hbm.at[wid], idx_v)
        col16 = lax.iota(jnp.int32, 16) * rstride

        def fire(g, slot):
            for k in range(_G // 16):
                sl = pl.ds(k * 16, 16)
                blk_idx[slot, sl] = lax.shift_right_logical(idx_v[g, sl], 3)
            pltpu.make_async_copy(
                er_hbm.at[blk_idx.at[slot]], blk_r.at[slot], sem_r).start()
            pltpu.make_async_copy(
                ei_hbm.at[blk_idx.at[slot]], blk_i.at[slot], sem_i).start()

        fire(0, 0)

        def body(g, carry):
            slot = lax.rem(g, 2)

            @pl.when(g + 1 < ng)
            def _():
                fire(g + 1, 1 - slot)

            pltpu.make_async_copy(
                er_hbm.at[blk_idx.at[slot]], blk_r.at[slot], sem_r).wait()
            pltpu.make_async_copy(
                ei_hbm.at[blk_idx.at[slot]], blk_i.at[slot], sem_i).wait()

            t_base = lax.rem(g, _GPC) * _G
            for t in range(_G // 16):
                idvec = idx_v[g, pl.ds(t * 16, 16)]
                col0v = (idvec & 7) * _DIM
                for lane in range(16):
                    i = t * 16 + lane
                    col0 = col0v[lane]
                    cols = col16 + (t_base + i)
                    plsc.store_scatter(
                        pr_v, [cols], blk_r[slot, i, pl.ds(col0, _DIM)])
                    plsc.store_scatter(
                        pi_v, [cols], blk_i[slot, i, pl.ds(col0, _DIM)])

            @pl.when(lax.rem(g, _GPC) == _GPC - 1)
            def _():
                u0 = wid * npw + (g // _GPC) * rows
                l_idx = u0 // b_batch
                b0 = lax.rem(u0, b_batch)
                owaits = []
                for d in range(_DIM):
                    src = pl.ds(d * rstride, rows)
                    o = (l_idx * _DIM + d) * b_batch + b0
                    owaits.append(pltpu.async_copy(
                        pr_v.at[src], out_r.at[pl.ds(o, rows)], sem_o))
                    owaits.append(pltpu.async_copy(
                        pi_v.at[src], out_i.at[pl.ds(o, rows)], sem_o))
                for w in owaits:
                    w.wait()

            return carry

        lax.fori_loop(0, ng, body, 0)

    return gather_kernel, nw, ng


def _blocked(table):
    return jnp.concatenate([table[i::8, :] for i in range(8)], axis=1)


def kernel(ids, embed, imag_embed):
    b, l = ids.shape
    vocab = embed.shape[0]
    gather_kernel, nw, ng = _build_gather(b, l, vocab)
    ids_u = ids.T.reshape(nw, ng, _G).astype(jnp.int32)
    er = _blocked(embed)
    ei = _blocked(imag_embed)
    out_r, out_i = gather_kernel(ids_u, er, ei)
    r_t, i_t = lax.optimization_barrier(
        (out_r.reshape(l, _DIM, b), out_i.reshape(l, _DIM, b)))
    c_t = lax.optimization_barrier(lax.complex(r_t, i_t))
    return lax.transpose(c_t, (2, 0, 1))


# speedup vs baseline: 4.3558x; 4.3558x over previous
import functools

import jax
import jax.numpy as jnp
from jax import lax
from jax.experimental import pallas as pl
from jax.experimental.pallas import tpu as pltpu
from jax.experimental.pallas import tpu_sc as plsc

_DIM = 16
_G = 128
_CH = 8


@functools.lru_cache(maxsize=None)
def _build_gather(b_batch: int, l_seq: int, vocab: int):
    info = plsc.get_sparse_core_info()
    nc, ns = info.num_cores, info.num_subcores
    nw = nc * ns
    total = b_batch * l_seq
    npw = total // nw
    rows = _CH * _G
    assert npw * nw == total and npw % rows == 0
    assert b_batch % rows == 0
    ng = npw // _G
    nchunk = ng // _CH

    mesh = plsc.VectorSubcoreMesh(core_axis_name="c", subcore_axis_name="s")

    @functools.partial(
        pl.kernel,
        mesh=mesh,
        compiler_params=pltpu.CompilerParams(
            use_tc_tiling_on_sc=False, needs_layout_passes=False),
        out_type=[
            jax.ShapeDtypeStruct((l_seq * _DIM * b_batch,), jnp.float32),
            jax.ShapeDtypeStruct((l_seq * _DIM * b_batch,), jnp.float32),
        ],
        scratch_types=[
            pltpu.VMEM((ng, _G), jnp.int32),
            pltpu.VMEM((2, rows, _DIM), jnp.float32),
            pltpu.VMEM((2, rows, _DIM), jnp.float32),
            pltpu.VMEM((_DIM * (rows + 8),), jnp.float32),
            pltpu.VMEM((_DIM * (rows + 8),), jnp.float32),
            pltpu.SemaphoreType.DMA,
            pltpu.SemaphoreType.DMA,
            pltpu.SemaphoreType.DMA,
        ],
    )
    def gather_kernel(ids_hbm, embed_hbm, imag_hbm, out_r, out_i,
                      idx_v, real_v, imag_v, pr_v, pi_v,
                      sem_r, sem_i, sem_o):
        wid = lax.axis_index("s") * nc + lax.axis_index("c")
        pltpu.sync_copy(ids_hbm.at[wid], idx_v)
        col16 = lax.iota(jnp.int32, 16) * (rows + 8)

        def fire(c, slot):
            for j in range(_CH):
                g = c * _CH + j
                dst = pl.ds(j * _G, _G)
                pltpu.make_async_copy(
                    embed_hbm.at[idx_v.at[g]],
                    real_v.at[slot].at[dst], sem_r).start()
                pltpu.make_async_copy(
                    imag_hbm.at[idx_v.at[g]],
                    imag_v.at[slot].at[dst], sem_i).start()

        def drain(c, slot):
            for j in range(_CH):
                g = c * _CH + j
                dst = pl.ds(j * _G, _G)
                pltpu.make_async_copy(
                    embed_hbm.at[idx_v.at[g]],
                    real_v.at[slot].at[dst], sem_r).wait()
                pltpu.make_async_copy(
                    imag_hbm.at[idx_v.at[g]],
                    imag_v.at[slot].at[dst], sem_i).wait()

        fire(0, 0)

        def chunk_body(c, carry):
            slot = lax.rem(c, 2)

            @pl.when(c + 1 < nchunk)
            def _():
                fire(c + 1, 1 - slot)

            drain(c, slot)

            def blk_body(blk, carry2):
                t0 = blk * 16
                for u in range(16):
                    cols = col16 + (t0 + u)
                    plsc.store_scatter(pr_v, [cols], real_v[slot, t0 + u, :])
                    plsc.store_scatter(pi_v, [cols], imag_v[slot, t0 + u, :])
                return carry2

            lax.fori_loop(0, rows // 16, blk_body, 0)

            u0 = wid * npw + c * rows
            l_idx = u0 // b_batch
            b0 = lax.rem(u0, b_batch)
            owaits = []
            for d in range(_DIM):
                src = pl.ds(d * (rows + 8), rows)
                o = (l_idx * _DIM + d) * b_batch + b0
                owaits.append(pltpu.async_copy(
                    pr_v.at[src], out_r.at[pl.ds(o, rows)], sem_o))
                owaits.append(pltpu.async_copy(
                    pi_v.at[src], out_i.at[pl.ds(o, rows)], sem_o))
            for w in owaits:
                w.wait()
            return carry

        lax.fori_loop(0, nchunk, chunk_body, 0)

    return gather_kernel, nw, ng


def kernel(ids, embed, imag_embed):
    b, l = ids.shape
    vocab = embed.shape[0]
    gather_kernel, nw, ng = _build_gather(b, l, vocab)
    ids_u = ids.T.reshape(nw, ng, _G).astype(jnp.int32)
    out_r, out_i = gather_kernel(ids_u, embed, imag_embed)
    r_t, i_t = lax.optimization_barrier(
        (out_r.reshape(l, _DIM, b), out_i.reshape(l, _DIM, b)))
    c_t = lax.optimization_barrier(lax.complex(r_t, i_t))
    return lax.transpose(c_t, (2, 0, 1))


# speedup vs baseline: 4.5605x; 1.0470x over previous
import functools

import jax
import jax.numpy as jnp
from jax import lax
from jax.experimental import pallas as pl
from jax.experimental.pallas import tpu as pltpu
from jax.experimental.pallas import tpu_sc as plsc

_DIM = 16
_G = 128
_CH = 8


@functools.lru_cache(maxsize=None)
def _build_gather(b_batch: int, l_seq: int, vocab: int):
    info = plsc.get_sparse_core_info()
    nc, ns = info.num_cores, info.num_subcores
    nw = nc * ns
    total = b_batch * l_seq
    npw = total // nw
    rows = _CH * _G
    assert npw * nw == total and npw % rows == 0
    assert b_batch % rows == 0
    ng = npw // _G
    nchunk = ng // _CH
    rstride = rows + 8

    mesh = plsc.VectorSubcoreMesh(core_axis_name="c", subcore_axis_name="s")

    @functools.partial(
        pl.kernel,
        mesh=mesh,
        compiler_params=pltpu.CompilerParams(
            use_tc_tiling_on_sc=False, needs_layout_passes=False),
        out_type=jax.ShapeDtypeStruct((l_seq * _DIM * b_batch,), jnp.float32),
        scratch_types=[
            pltpu.VMEM((ng, _G), jnp.int32),
            pltpu.VMEM((2, rows, _DIM), jnp.float32),
            pltpu.VMEM((_DIM * rstride,), jnp.float32),
            pltpu.SemaphoreType.DMA,
            pltpu.SemaphoreType.DMA,
        ],
    )
    def gather_kernel(ids_hbm, table_hbm, out_p,
                      idx_v, row_v, pl_v, sem_g, sem_o):
        wid = lax.axis_index("s") * nc + lax.axis_index("c")
        pltpu.sync_copy(ids_hbm.at[wid], idx_v)
        col16 = lax.iota(jnp.int32, 16) * rstride

        def fire(c, slot):
            for j in range(_CH):
                g = c * _CH + j
                dst = pl.ds(j * _G, _G)
                pltpu.make_async_copy(
                    table_hbm.at[idx_v.at[g]],
                    row_v.at[slot].at[dst], sem_g).start()

        def drain(c, slot):
            for j in range(_CH):
                g = c * _CH + j
                dst = pl.ds(j * _G, _G)
                pltpu.make_async_copy(
                    table_hbm.at[idx_v.at[g]],
                    row_v.at[slot].at[dst], sem_g).wait()

        fire(0, 0)

        def chunk_body(c, carry):
            slot = lax.rem(c, 2)

            @pl.when(c + 1 < nchunk)
            def _():
                fire(c + 1, 1 - slot)

            drain(c, slot)

            def blk_body(blk, carry2):
                t0 = blk * 16
                for u in range(16):
                    cols = col16 + (t0 + u)
                    plsc.store_scatter(pl_v, [cols], row_v[slot, t0 + u, :])
                return carry2

            lax.fori_loop(0, rows // 16, blk_body, 0)

            u0 = wid * npw + c * rows
            l_idx = u0 // b_batch
            b0 = lax.rem(u0, b_batch)
            owaits = []
            for d in range(_DIM):
                src = pl.ds(d * rstride, rows)
                o = (l_idx * _DIM + d) * b_batch + b0
                owaits.append(pltpu.async_copy(
                    pl_v.at[src], out_p.at[pl.ds(o, rows)], sem_o))
            for w in owaits:
                w.wait()
            return carry

        lax.fori_loop(0, nchunk, chunk_body, 0)

    return gather_kernel, nw, ng


def kernel(ids, embed, imag_embed):
    b, l = ids.shape
    vocab = embed.shape[0]
    gather_kernel, nw, ng = _build_gather(b, l, vocab)
    ids_u = ids.T.reshape(nw, ng, _G).astype(jnp.int32)
    out_r = gather_kernel(ids_u, embed)
    out_i = gather_kernel(ids_u, imag_embed)
    r_t, i_t = lax.optimization_barrier(
        (out_r.reshape(l, _DIM, b), out_i.reshape(l, _DIM, b)))
    c_t = lax.optimization_barrier(lax.complex(r_t, i_t))
    return lax.transpose(c_t, (2, 0, 1))
